# baseline (speedup 1.0000x reference)
import functools

import jax
import jax.numpy as jnp
from jax import lax
from jax.experimental import pallas as pl
from jax.experimental.pallas import tpu as pltpu
from jax.experimental.pallas import tpu_sc as plsc

NC = 2
NS = 16
NW = NC * NS
CHUNK = 128


def _make_sc_aggregate(n_acc, d, nch0, nch1, with_deg):
    rpt = n_acc // NS
    grp = 8
    assert nch0 % (2 * grp) == 0 and nch1 % (2 * grp) == 0
    n_slab = 1 if nch1 == 0 else NC
    xr = 32
    nxb = rpt // xr
    mesh = plsc.VectorSubcoreMesh(core_axis_name="c", subcore_axis_name="s",
                                  num_cores=NC, num_subcores=NS)

    out_type = [jax.ShapeDtypeStruct((n_slab, n_acc, d), jnp.float32)]
    scratch = [
        pltpu.VMEM_SHARED((n_acc, d), jnp.float32),
        pltpu.VMEM((grp, CHUNK), jnp.int32),
        pltpu.VMEM((grp, CHUNK), jnp.int32),
        pltpu.VMEM((grp, CHUNK), jnp.int32),
        pltpu.VMEM((grp, CHUNK), jnp.int32),
        pltpu.VMEM((CHUNK, d), jnp.float32),
        pltpu.VMEM((CHUNK, d), jnp.float32),
        pltpu.VMEM((xr, d), jnp.float32),
        pltpu.SemaphoreType.DMA,
        pltpu.SemaphoreType.DMA,
    ]
    if with_deg:
        out_type.append(jax.ShapeDtypeStruct((n_slab, n_acc), jnp.float32))
        scratch += [
            pltpu.VMEM_SHARED((n_acc,), jnp.float32),
            pltpu.VMEM((CHUNK,), jnp.float32),
            pltpu.VMEM((rpt,), jnp.float32),
        ]

    def body(table, srcp, dstp, zeros_d, zeros_g, ones_g, *rest):
        if with_deg:
            (part, degp, acc, src0, src1, dst0, dst1, rows0, rows1, xbuf,
             gsem, sem, dega, ones_v, gbuf) = rest
        else:
            (part, acc, src0, src1, dst0, dst1, rows0, rows1, xbuf,
             gsem, sem) = rest
        src_st = (src0, src1)
        dst_st = (dst0, dst1)
        rows = (rows0, rows1)
        c = lax.axis_index("c")
        s = lax.axis_index("s")
        r0 = s * rpt
        g0 = jnp.where(c == 0, s * nch0, NS * nch0 + s * nch1)
        n_pair = jnp.where(c == 0, nch0 // (2 * grp), nch1 // (2 * grp))

        slab = c if n_slab == NC else 0

        def prologue():
            pltpu.sync_copy(zeros_d, xbuf)
            for j in range(nxb):
                pltpu.sync_copy(xbuf, acc.at[pl.ds(r0 + j * xr, xr), :])
            if with_deg:
                pltpu.sync_copy(zeros_g, gbuf)
                pltpu.sync_copy(gbuf, dega.at[pl.ds(r0, rpt)])
                pltpu.sync_copy(ones_g, ones_v)
            pltpu.sync_copy(srcp.at[pl.ds(g0, grp), :], src_st[0])
            pltpu.sync_copy(dstp.at[pl.ds(g0, grp), :], dst_st[0])
            pltpu.async_copy(table.at[src_st[0].at[0]], rows[0], gsem)

        if n_slab == NC:
            prologue()
        else:
            pl.when(c == 0)(prologue)
        plsc.subcore_barrier()

        def wait_gather(buf):
            pltpu.make_async_copy(table.at[pl.ds(0, CHUNK)], buf, gsem).wait()

        def pair(g2, _):
            for gg in range(2):
                g = g2 * 2 + gg
                cur_s, cur_d = src_st[gg], dst_st[gg]
                nxt_s, nxt_d = src_st[1 - gg], dst_st[1 - gg]
                is_last = g2 == n_pair - 1 if gg else None
                if gg == 0:
                    pltpu.sync_copy(srcp.at[pl.ds(g0 + (g + 1) * grp, grp), :],
                                    nxt_s)
                    pltpu.sync_copy(dstp.at[pl.ds(g0 + (g + 1) * grp, grp), :],
                                    nxt_d)
                else:
                    @pl.when(jnp.logical_not(is_last))
                    def _stage():
                        pltpu.sync_copy(
                            srcp.at[pl.ds(g0 + (g + 1) * grp, grp), :], nxt_s)
                        pltpu.sync_copy(
                            dstp.at[pl.ds(g0 + (g + 1) * grp, grp), :], nxt_d)
                for j in range(grp):
                    b = j % 2
                    wait_gather(rows[b])
                    if j < grp - 1:
                        pltpu.async_copy(table.at[cur_s.at[j + 1]],
                                         rows[1 - b], gsem)
                    elif gg == 0:
                        pltpu.async_copy(table.at[nxt_s.at[0]],
                                         rows[1 - b], gsem)
                    else:
                        @pl.when(jnp.logical_not(is_last))
                        def _fire():
                            pltpu.async_copy(table.at[nxt_s.at[0]],
                                             rows[1 - b], gsem)
                    pltpu.async_copy(rows[b], acc.at[cur_d.at[j]], sem,
                                     add=True).wait()
                    if with_deg:
                        pltpu.async_copy(ones_v, dega.at[cur_d.at[j]], sem,
                                         add=True).wait()
            return _

        lax.fori_loop(0, n_pair, pair, None)
        plsc.subcore_barrier()

        def writeback():
            for j in range(nxb):
                sl = pl.ds(r0 + j * xr, xr)
                pltpu.sync_copy(acc.at[sl, :], xbuf)
                pltpu.sync_copy(xbuf, part.at[slab, sl, :])
            if with_deg:
                pltpu.sync_copy(dega.at[pl.ds(r0, rpt)], gbuf)
                pltpu.sync_copy(gbuf, degp.at[slab, pl.ds(r0, rpt)])

        if n_slab == NC:
            writeback()
        else:
            pl.when(c == 0)(writeback)

    return pl.kernel(body, out_type=tuple(out_type), mesh=mesh,
                     scratch_types=scratch)


def _lin2_body(x_ref, wl_ref, wr_ref, b_ref, y_ref, z_ref):
    x = x_ref[...]
    dn = (((1,), (1,)), ((), ()))
    y_ref[...] = lax.dot_general(x, wl_ref[...], dn,
                                 preferred_element_type=jnp.float32)
    z_ref[...] = lax.dot_general(x, wr_ref[...], dn,
                                 preferred_element_type=jnp.float32) \
        + b_ref[...][None, :]


def _relu_stats_body(n_valid, blk, p_ref, degp_ref, z_ref, h_ref, st_ref):
    i = pl.program_id(0)
    deg = jnp.maximum(jnp.sum(degp_ref[...], axis=0), 1.0)
    h = jnp.maximum(jnp.sum(p_ref[...], axis=0) / deg[:, None] + z_ref[...],
                    0.0)
    h_ref[...] = h

    @pl.when(i == 0)
    def _init():
        st_ref[...] = jnp.zeros_like(st_ref)

    row = i * blk + lax.broadcasted_iota(jnp.int32, (blk, 1), 0)
    hm = jnp.where(row < n_valid, h, 0.0)
    st_ref[0, :] += jnp.sum(hm, axis=0)
    st_ref[1, :] += jnp.sum(hm * hm, axis=0)


def _final_body(n_rows, h_ref, q_ref, degp_ref, st_ref, gamma_ref, beta_ref,
                wl_ref, wr_ref, b_ref, o_ref):
    inv_n = 1.0 / n_rows
    mu = st_ref[0, :] * inv_n
    var = st_ref[1, :] * inv_n - mu * mu
    sc = gamma_ref[...] * lax.rsqrt(var + 1e-5)
    t = beta_ref[...] - mu * sc
    deg = jnp.maximum(jnp.sum(degp_ref[...], axis=0), 1.0)
    mn = jnp.sum(q_ref[...], axis=0) / deg[:, None]
    mn = mn * sc[None, :] + t[None, :]
    hn = h_ref[...] * sc[None, :] + t[None, :]
    dn = (((1,), (1,)), ((), ()))
    o = lax.dot_general(mn, wl_ref[...], dn,
                        preferred_element_type=jnp.float32) \
        + lax.dot_general(hn, wr_ref[...], dn,
                          preferred_element_type=jnp.float32) \
        + b_ref[...][None, :]
    m = jnp.max(o, axis=1, keepdims=True)
    lse = jnp.log(jnp.sum(jnp.exp(o - m), axis=1, keepdims=True))
    o_ref[...] = o - m - lse


def kernel(x, edge_index, Wl1, b1, Wr1, gamma1, beta1, Wl2, b2, Wr2):
    n, din = x.shape
    dh = Wl1.shape[0]
    dout = Wl2.shape[0]
    e = edge_index.shape[1]

    blk = 1024
    n_acc = ((n // blk) + 1) * blk
    grid = n_acc // blk
    quantum = CHUNK * 8 * 2
    ept = ((e // NW + quantum - 1) // quantum) * quantum
    epad = ept * NW
    rpt = n_acc // NS
    nch_tot = epad // CHUNK // NS
    nch0 = nch_tot
    nch1 = nch_tot - nch0
    assert nch0 % 16 == 0 and nch1 % 16 == 0
    n_slab = 1 if nch1 == 0 else NC

    src = edge_index[0]
    dst = edge_index[1]
    pad = epad - e
    srcp = jnp.concatenate([src, jnp.zeros((pad,), jnp.int32)])
    dummy_dst = n + jnp.arange(pad, dtype=jnp.int32) % (n_acc - n)
    dstp = jnp.concatenate([dst, dummy_dst])
    srcp = srcp.reshape(-1, CHUNK)
    dstp = dstp.reshape(-1, CHUNK)
    xp = jnp.concatenate([x, jnp.zeros((n_acc - n, din), jnp.float32)])
    zeros_d = jnp.zeros((32, dh), jnp.float32)
    zeros_g = jnp.zeros((rpt,), jnp.float32)
    ones_g = jnp.ones((CHUNK,), jnp.float32)

    f32 = jnp.float32
    row_spec = pl.BlockSpec((blk, dh), lambda i: (i, 0))
    par_spec = pl.BlockSpec((n_slab, blk, dh), lambda i: (0, i, 0))
    deg_spec = pl.BlockSpec((n_slab, blk), lambda i: (0, i))

    y1, z1 = pl.pallas_call(
        _lin2_body,
        grid=(grid,),
        in_specs=[pl.BlockSpec((blk, din), lambda i: (i, 0)),
                  pl.BlockSpec((dh, din), lambda i: (0, 0)),
                  pl.BlockSpec((dh, din), lambda i: (0, 0)),
                  pl.BlockSpec((dh,), lambda i: (0,))],
        out_specs=[row_spec, row_spec],
        out_shape=[jax.ShapeDtypeStruct((n_acc, dh), f32),
                   jax.ShapeDtypeStruct((n_acc, dh), f32)],
    )(xp, Wl1, Wr1, b1)

    agg1 = _make_sc_aggregate(n_acc, dh, nch0, nch1, with_deg=True)
    part1, degp = agg1(y1, srcp, dstp, zeros_d, zeros_g, ones_g)

    h, stats = pl.pallas_call(
        functools.partial(_relu_stats_body, n, blk),
        grid=(grid,),
        in_specs=[par_spec, deg_spec, row_spec],
        out_specs=[row_spec, pl.BlockSpec((2, dh), lambda i: (0, 0))],
        out_shape=[jax.ShapeDtypeStruct((n_acc, dh), f32),
                   jax.ShapeDtypeStruct((2, dh), f32)],
    )(part1, degp, z1)

    agg2 = _make_sc_aggregate(n_acc, dh, nch0, nch1, with_deg=False)
    (part2,) = agg2(h, srcp, dstp, zeros_d, zeros_g, ones_g)

    out = pl.pallas_call(
        functools.partial(_final_body, float(n)),
        grid=(grid,),
        in_specs=[row_spec, par_spec, deg_spec,
                  pl.BlockSpec((2, dh), lambda i: (0, 0)),
                  pl.BlockSpec((dh,), lambda i: (0,)),
                  pl.BlockSpec((dh,), lambda i: (0,)),
                  pl.BlockSpec((dout, dh), lambda i: (0, 0)),
                  pl.BlockSpec((dout, dh), lambda i: (0, 0)),
                  pl.BlockSpec((dout,), lambda i: (0,))],
        out_specs=pl.BlockSpec((blk, dout), lambda i: (i, 0)),
        out_shape=jax.ShapeDtypeStruct((n_acc, dout), f32),
    )(h, part2, degp, stats, gamma1, beta1, Wl2, Wr2, b2)
    return out[:n]


# speedup vs baseline: 1.3588x; 1.3588x over previous
import functools

import jax
import jax.numpy as jnp
from jax import lax
from jax.experimental import pallas as pl
from jax.experimental.pallas import tpu as pltpu
from jax.experimental.pallas import tpu_sc as plsc

NC = 2
NS = 16
NW = NC * NS
CHUNK = 128


def _make_sc_aggregate(n_acc, d, nch0, nch1, with_deg):
    rpt = n_acc // NS
    grp = 8
    assert nch0 % (2 * grp) == 0 and nch1 % (2 * grp) == 0
    n_slab = 1 if nch1 == 0 else NC
    xr = 32
    nxb = rpt // xr
    mesh = plsc.VectorSubcoreMesh(core_axis_name="c", subcore_axis_name="s",
                                  num_cores=NC, num_subcores=NS)

    out_type = [jax.ShapeDtypeStruct((n_slab, n_acc, d), jnp.float32)]
    scratch = [
        pltpu.VMEM_SHARED((n_acc, d), jnp.float32),
        pltpu.VMEM((grp, CHUNK), jnp.int32),
        pltpu.VMEM((grp, CHUNK), jnp.int32),
        pltpu.VMEM((grp, CHUNK), jnp.int32),
        pltpu.VMEM((grp, CHUNK), jnp.int32),
        pltpu.VMEM((CHUNK, d), jnp.float32),
        pltpu.VMEM((CHUNK, d), jnp.float32),
        pltpu.VMEM((xr, d), jnp.float32),
        pltpu.SemaphoreType.DMA,
        pltpu.SemaphoreType.DMA,
    ]
    if with_deg:
        out_type.append(jax.ShapeDtypeStruct((n_slab, n_acc), jnp.float32))
        scratch += [
            pltpu.VMEM_SHARED((n_acc,), jnp.float32),
            pltpu.VMEM((CHUNK,), jnp.float32),
            pltpu.VMEM((rpt,), jnp.float32),
        ]

    def body(table, srcp, dstp, zeros_d, zeros_g, ones_g, *rest):
        if with_deg:
            (part, degp, acc, src0, src1, dst0, dst1, rows0, rows1, xbuf,
             gsem, sem, dega, ones_v, gbuf) = rest
        else:
            (part, acc, src0, src1, dst0, dst1, rows0, rows1, xbuf,
             gsem, sem) = rest
        src_st = (src0, src1)
        dst_st = (dst0, dst1)
        rows = (rows0, rows1)
        c = lax.axis_index("c")
        s = lax.axis_index("s")
        r0 = s * rpt
        g0 = jnp.where(c == 0, s * nch0, NS * nch0 + s * nch1)
        n_pair = jnp.where(c == 0, nch0 // (2 * grp), nch1 // (2 * grp))

        slab = c if n_slab == NC else 0

        def prologue():
            pltpu.sync_copy(zeros_d, xbuf)
            for j in range(nxb):
                pltpu.sync_copy(xbuf, acc.at[pl.ds(r0 + j * xr, xr), :])
            if with_deg:
                pltpu.sync_copy(zeros_g, gbuf)
                pltpu.sync_copy(gbuf, dega.at[pl.ds(r0, rpt)])
                pltpu.sync_copy(ones_g, ones_v)
            pltpu.sync_copy(srcp.at[pl.ds(g0, grp), :], src_st[0])
            pltpu.sync_copy(dstp.at[pl.ds(g0, grp), :], dst_st[0])
            pltpu.async_copy(table.at[src_st[0].at[0]], rows[0], gsem)

        if n_slab == NC:
            prologue()
        else:
            pl.when(c == 0)(prologue)
        plsc.subcore_barrier()

        def wait_gather(buf):
            pltpu.make_async_copy(table.at[pl.ds(0, CHUNK)], buf, gsem).wait()

        def pair(g2, _):
            for gg in range(2):
                g = g2 * 2 + gg
                cur_s, cur_d = src_st[gg], dst_st[gg]
                nxt_s, nxt_d = src_st[1 - gg], dst_st[1 - gg]
                is_last = g2 == n_pair - 1 if gg else None
                if gg == 0:
                    pltpu.sync_copy(srcp.at[pl.ds(g0 + (g + 1) * grp, grp), :],
                                    nxt_s)
                    pltpu.sync_copy(dstp.at[pl.ds(g0 + (g + 1) * grp, grp), :],
                                    nxt_d)
                else:
                    @pl.when(jnp.logical_not(is_last))
                    def _stage():
                        pltpu.sync_copy(
                            srcp.at[pl.ds(g0 + (g + 1) * grp, grp), :], nxt_s)
                        pltpu.sync_copy(
                            dstp.at[pl.ds(g0 + (g + 1) * grp, grp), :], nxt_d)
                for j in range(grp):
                    b = j % 2
                    wait_gather(rows[b])
                    if j < grp - 1:
                        pltpu.async_copy(table.at[cur_s.at[j + 1]],
                                         rows[1 - b], gsem)
                    elif gg == 0:
                        pltpu.async_copy(table.at[nxt_s.at[0]],
                                         rows[1 - b], gsem)
                    else:
                        @pl.when(jnp.logical_not(is_last))
                        def _fire():
                            pltpu.async_copy(table.at[nxt_s.at[0]],
                                             rows[1 - b], gsem)
                    pltpu.async_copy(rows[b], acc.at[cur_d.at[j]], sem,
                                     add=True).wait()
                    if with_deg:
                        pltpu.async_copy(ones_v, dega.at[cur_d.at[j]], sem,
                                         add=True).wait()
            return _

        lax.fori_loop(0, n_pair, pair, None)
        plsc.subcore_barrier()

        def writeback():
            for j in range(nxb):
                sl = pl.ds(r0 + j * xr, xr)
                pltpu.sync_copy(acc.at[sl, :], xbuf)
                pltpu.sync_copy(xbuf, part.at[slab, sl, :])
            if with_deg:
                pltpu.sync_copy(dega.at[pl.ds(r0, rpt)], gbuf)
                pltpu.sync_copy(gbuf, degp.at[slab, pl.ds(r0, rpt)])

        if n_slab == NC:
            writeback()
        else:
            pl.when(c == 0)(writeback)

    return pl.kernel(body, out_type=tuple(out_type), mesh=mesh,
                     scratch_types=scratch)


def _lin2_body(x_ref, wl_ref, wr_ref, b_ref, y_ref, z_ref):
    x = x_ref[...]
    dn = (((1,), (1,)), ((), ()))
    y_ref[...] = lax.dot_general(x, wl_ref[...], dn,
                                 preferred_element_type=jnp.float32)
    z_ref[...] = lax.dot_general(x, wr_ref[...], dn,
                                 preferred_element_type=jnp.float32) \
        + b_ref[...][None, :]


def _relu_stats_body(n_valid, blk, p_ref, degp_ref, z_ref, h_ref, st_ref):
    i = pl.program_id(0)
    deg = jnp.maximum(jnp.sum(degp_ref[...], axis=0), 1.0)
    h = jnp.maximum(jnp.sum(p_ref[...], axis=0) / deg[:, None] + z_ref[...],
                    0.0)
    h_ref[...] = h

    @pl.when(i == 0)
    def _init():
        st_ref[...] = jnp.zeros_like(st_ref)

    row = i * blk + lax.broadcasted_iota(jnp.int32, (blk, 1), 0)
    hm = jnp.where(row < n_valid, h, 0.0)
    st_ref[0, :] += jnp.sum(hm, axis=0)
    st_ref[1, :] += jnp.sum(hm * hm, axis=0)


def _final_body(n_rows, h_ref, q_ref, degp_ref, st_ref, gamma_ref, beta_ref,
                wl_ref, wr_ref, b_ref, o_ref):
    inv_n = 1.0 / n_rows
    mu = st_ref[0, :] * inv_n
    var = st_ref[1, :] * inv_n - mu * mu
    sc = gamma_ref[...] * lax.rsqrt(var + 1e-5)
    t = beta_ref[...] - mu * sc
    deg = jnp.maximum(jnp.sum(degp_ref[...], axis=0), 1.0)
    mn = jnp.sum(q_ref[...], axis=0) / deg[:, None]
    mn = mn * sc[None, :] + t[None, :]
    hn = h_ref[...] * sc[None, :] + t[None, :]
    dn = (((1,), (1,)), ((), ()))
    o = lax.dot_general(mn, wl_ref[...], dn,
                        preferred_element_type=jnp.float32) \
        + lax.dot_general(hn, wr_ref[...], dn,
                          preferred_element_type=jnp.float32) \
        + b_ref[...][None, :]
    m = jnp.max(o, axis=1, keepdims=True)
    lse = jnp.log(jnp.sum(jnp.exp(o - m), axis=1, keepdims=True))
    o_ref[...] = o - m - lse


def kernel(x, edge_index, Wl1, b1, Wr1, gamma1, beta1, Wl2, b2, Wr2):
    n, din = x.shape
    dh = Wl1.shape[0]
    dout = Wl2.shape[0]
    e = edge_index.shape[1]

    blk = 1024
    n_acc = ((n // blk) + 1) * blk
    grid = n_acc // blk
    quantum = CHUNK * 8 * 2
    ept = ((e // NW + quantum - 1) // quantum) * quantum
    epad = ept * NW
    rpt = n_acc // NS
    nch_tot = epad // CHUNK // NS
    nch0 = (nch_tot * 9 // 10) // 16 * 16
    nch1 = nch_tot - nch0
    assert nch0 % 16 == 0 and nch1 % 16 == 0
    n_slab = 1 if nch1 == 0 else NC

    src = edge_index[0]
    dst = edge_index[1]
    pad = epad - e
    srcp = jnp.concatenate([src, jnp.zeros((pad,), jnp.int32)])
    dummy_dst = n + jnp.arange(pad, dtype=jnp.int32) % (n_acc - n)
    dstp = jnp.concatenate([dst, dummy_dst])
    srcp = srcp.reshape(-1, CHUNK)
    dstp = dstp.reshape(-1, CHUNK)
    xp = jnp.concatenate([x, jnp.zeros((n_acc - n, din), jnp.float32)])
    zeros_d = jnp.zeros((32, dh), jnp.float32)
    zeros_g = jnp.zeros((rpt,), jnp.float32)
    ones_g = jnp.ones((CHUNK,), jnp.float32)

    f32 = jnp.float32
    row_spec = pl.BlockSpec((blk, dh), lambda i: (i, 0))
    par_spec = pl.BlockSpec((n_slab, blk, dh), lambda i: (0, i, 0))
    deg_spec = pl.BlockSpec((n_slab, blk), lambda i: (0, i))

    y1, z1 = pl.pallas_call(
        _lin2_body,
        grid=(grid,),
        in_specs=[pl.BlockSpec((blk, din), lambda i: (i, 0)),
                  pl.BlockSpec((dh, din), lambda i: (0, 0)),
                  pl.BlockSpec((dh, din), lambda i: (0, 0)),
                  pl.BlockSpec((dh,), lambda i: (0,))],
        out_specs=[row_spec, row_spec],
        out_shape=[jax.ShapeDtypeStruct((n_acc, dh), f32),
                   jax.ShapeDtypeStruct((n_acc, dh), f32)],
    )(xp, Wl1, Wr1, b1)

    agg1 = _make_sc_aggregate(n_acc, dh, nch0, nch1, with_deg=True)
    part1, degp = agg1(y1, srcp, dstp, zeros_d, zeros_g, ones_g)

    h, stats = pl.pallas_call(
        functools.partial(_relu_stats_body, n, blk),
        grid=(grid,),
        in_specs=[par_spec, deg_spec, row_spec],
        out_specs=[row_spec, pl.BlockSpec((2, dh), lambda i: (0, 0))],
        out_shape=[jax.ShapeDtypeStruct((n_acc, dh), f32),
                   jax.ShapeDtypeStruct((2, dh), f32)],
    )(part1, degp, z1)

    agg2 = _make_sc_aggregate(n_acc, dh, nch0, nch1, with_deg=False)
    (part2,) = agg2(h, srcp, dstp, zeros_d, zeros_g, ones_g)

    out = pl.pallas_call(
        functools.partial(_final_body, float(n)),
        grid=(grid,),
        in_specs=[row_spec, par_spec, deg_spec,
                  pl.BlockSpec((2, dh), lambda i: (0, 0)),
                  pl.BlockSpec((dh,), lambda i: (0,)),
                  pl.BlockSpec((dh,), lambda i: (0,)),
                  pl.BlockSpec((dout, dh), lambda i: (0, 0)),
                  pl.BlockSpec((dout, dh), lambda i: (0, 0)),
                  pl.BlockSpec((dout,), lambda i: (0,))],
        out_specs=pl.BlockSpec((blk, dout), lambda i: (i, 0)),
        out_shape=jax.ShapeDtypeStruct((n_acc, dout), f32),
    )(h, part2, degp, stats, gamma1, beta1, Wl2, Wr2, b2)
    return out[:n]
